# baseline (speedup 1.0000x reference)
import jax
import jax.numpy as jnp
from jax.experimental import pallas as pl
from jax.experimental.pallas import tpu as pltpu
from jax.experimental.pallas import tpu_sc as plsc

B, HW, D, K = 8, 1024, 256, 8192
KC = 2048
GW = 128


LANES = 128


def _argmin_body(x_ref, e_ref, c_ref, idx_ref, et_ref):
    b = pl.program_id(0)
    x = x_ref[0]
    c = c_ref[0]
    cb = jnp.broadcast_to(c, (HW, LANES))
    lane = jax.lax.broadcasted_iota(jnp.int32, (HW, LANES), 1)
    ea = e_ref[0]
    s = jnp.sum(ea * ea, axis=0, keepdims=True)
    xm2 = x * (-2.0)
    vals = None
    cols = None
    for j in range(K // KC):
        e = e_ref[0, :, j * KC:(j + 1) * KC]
        m = jax.lax.dot_general(
            xm2, e, (((1,), (0,)), ((), ())),
            preferred_element_type=jnp.float32)
        for i in range(KC // LANES):
            col = j * (KC // LANES) + i
            sc = s[:, col * LANES:(col + 1) * LANES]
            scb = jnp.broadcast_to(sc, (HW, LANES))
            mc = m[:, i * LANES:(i + 1) * LANES]
            dc = (cb + mc) + scb
            if col == 0:
                vals = dc
                cols = jnp.zeros((HW, LANES), dtype=jnp.int32)
            else:
                lt = dc < vals
                cols = jnp.where(lt, jnp.int32(col), cols)
                vals = jnp.where(lt, dc, vals)
        et_ref[0, j * KC:(j + 1) * KC, :] = e.T
    idxs = cols * LANES + lane
    vmin = jnp.min(vals, axis=1, keepdims=True)
    amin = jnp.min(jnp.where(vals == vmin, idxs, K), axis=1, keepdims=True)
    idx_ref[0] = (amin + b * K).reshape(1, HW)


def _vq_argmin(inputs, embeddings, c):
    nb = inputs.shape[0]
    return pl.pallas_call(
        _argmin_body,
        grid=(nb,),
        in_specs=[
            pl.BlockSpec((1, HW, D), lambda b: (b, 0, 0)),
            pl.BlockSpec((1, D, K), lambda b: (b, 0, 0)),
            pl.BlockSpec((1, HW, 1), lambda b: (b, 0, 0)),
        ],
        out_specs=[
            pl.BlockSpec((1, 1, HW), lambda b: (b, 0, 0)),
            pl.BlockSpec((1, K, D), lambda b: (b, 0, 0)),
        ],
        out_shape=[
            jax.ShapeDtypeStruct((nb, 1, HW), jnp.int32),
            jax.ShapeDtypeStruct((nb, K, D), jnp.float32),
        ],
    )(inputs, embeddings, c)


def _sc_gather(et_flat, idx_flat):
    n_idx = idx_flat.shape[1]
    mesh = plsc.VectorSubcoreMesh(core_axis_name="core",
                                  subcore_axis_name="subcore")

    @pl.kernel(out_type=jax.ShapeDtypeStruct((n_idx, D), jnp.float32),
               mesh=mesh)
    def kern(x_hbm, i_hbm, o_hbm):
        def body(i_vmem, o_vmem):
            pltpu.sync_copy(x_hbm.at[i_vmem.at[0]], o_vmem)

        pltpu.emit_pipeline(
            body,
            grid=(n_idx // GW,),
            in_specs=[pl.BlockSpec((1, GW), lambda i: (0, i))],
            out_specs=[pl.BlockSpec((GW, D), lambda i: (i, 0))],
            core_axis_name=("core", "subcore"),
            dimension_semantics=(pltpu.PARALLEL,),
        )(i_hbm, o_hbm)

    return kern(et_flat, idx_flat)


def kernel(inputs, embeddings):
    c = jnp.sum(inputs ** 2, axis=2, keepdims=True)
    hb = B // 2
    outs = []
    for half in range(2):
        lo, hi = half * hb, (half + 1) * hb
        idx, et = _vq_argmin(inputs[lo:hi], embeddings[lo:hi], c[lo:hi])
        q = _sc_gather(et.reshape(hb * K, D), idx.reshape(1, hb * HW))
        outs.append(q.reshape(hb, HW, D))
    return jnp.concatenate(outs, axis=0)


# speedup vs baseline: 1.4738x; 1.4738x over previous
import jax
import jax.numpy as jnp
from jax.experimental import pallas as pl
from jax.experimental.pallas import tpu as pltpu
from jax.experimental.pallas import tpu_sc as plsc

B, HW, D, K = 8, 1024, 256, 8192
KC = 2048
GW = 128


LANES = 128


def _argmin_body(x_ref, e_ref, c_ref, idx_ref, et_ref):
    b = pl.program_id(0)
    x = x_ref[0]
    c = c_ref[0]
    cb = jnp.broadcast_to(c, (HW, LANES))
    lane = jax.lax.broadcasted_iota(jnp.int32, (HW, LANES), 1)
    ea = e_ref[0]
    s = jnp.sum(ea * ea, axis=0, keepdims=True)
    xm2 = x * (-2.0)
    vals = None
    cols = None
    for j in range(K // KC):
        e = e_ref[0, :, j * KC:(j + 1) * KC]
        m = jax.lax.dot_general(
            xm2, e, (((1,), (0,)), ((), ())),
            preferred_element_type=jnp.float32)
        for i in range(KC // LANES):
            col = j * (KC // LANES) + i
            sc = s[:, col * LANES:(col + 1) * LANES]
            scb = jnp.broadcast_to(sc, (HW, LANES))
            mc = m[:, i * LANES:(i + 1) * LANES]
            dc = (cb + mc) + scb
            if col == 0:
                vals = dc
                cols = jnp.zeros((HW, LANES), dtype=jnp.int32)
            else:
                lt = dc < vals
                cols = jnp.where(lt, jnp.int32(col), cols)
                vals = jnp.where(lt, dc, vals)
        et_ref[0, j * KC:(j + 1) * KC, :] = e.T
    idxs = cols * LANES + lane
    vmin = jnp.min(vals, axis=1, keepdims=True)
    amin = jnp.min(jnp.where(vals == vmin, idxs, K), axis=1, keepdims=True)
    idx_ref[0] = (amin + b * K).reshape(1, HW)


def _vq_argmin(inputs, embeddings, c):
    return pl.pallas_call(
        _argmin_body,
        grid=(B,),
        in_specs=[
            pl.BlockSpec((1, HW, D), lambda b: (b, 0, 0)),
            pl.BlockSpec((1, D, K), lambda b: (b, 0, 0)),
            pl.BlockSpec((1, HW, 1), lambda b: (b, 0, 0)),
        ],
        out_specs=[
            pl.BlockSpec((1, 1, HW), lambda b: (b, 0, 0)),
            pl.BlockSpec((1, K, D), lambda b: (b, 0, 0)),
        ],
        out_shape=[
            jax.ShapeDtypeStruct((B, 1, HW), jnp.int32),
            jax.ShapeDtypeStruct((B, K, D), jnp.float32),
        ],
    )(inputs, embeddings, c)


def _sc_gather(et_flat, idx_flat):
    mesh = plsc.VectorSubcoreMesh(core_axis_name="core",
                                  subcore_axis_name="subcore")

    @pl.kernel(out_type=jax.ShapeDtypeStruct((B * HW, D), jnp.float32),
               mesh=mesh)
    def kern(x_hbm, i_hbm, o_hbm):
        def body(i_vmem, o_vmem):
            pltpu.sync_copy(x_hbm.at[i_vmem.at[0]], o_vmem)

        pltpu.emit_pipeline(
            body,
            grid=(B * HW // GW,),
            in_specs=[pl.BlockSpec((1, GW), lambda i: (0, i))],
            out_specs=[pl.BlockSpec((GW, D), lambda i: (i, 0))],
            core_axis_name=("core", "subcore"),
            dimension_semantics=(pltpu.PARALLEL,),
        )(i_hbm, o_hbm)

    return kern(et_flat, idx_flat)


def kernel(inputs, embeddings):
    c = jnp.sum(inputs ** 2, axis=2, keepdims=True)
    idx, et = _vq_argmin(inputs, embeddings, c)
    idx_flat = idx.reshape(1, B * HW)
    et_flat = et.reshape(B * K, D)
    quantized = _sc_gather(et_flat, idx_flat).reshape(B, HW, D)
    return quantized
